# Initial kernel scaffold; baseline (speedup 1.0000x reference)
import jax
import jax.numpy as jnp
from jax.experimental import pallas as pl


def kernel(x_mrna, edge_index_mrna, W1_mrna, b1_mrna, W2_mrna, b2_mrna, x_meth, edge_index_meth, W1_meth, b1_meth, W2_meth, b2_meth, x_mirna, edge_index_mirna, W1_mirna, b1_mirna, W2_mirna, b2_mirna, Wi1, bi1, Wi2, bi2):
    raise NotImplementedError("write your pallas kernel here")


# speedup vs baseline: 6.4639x; 6.4639x over previous
import functools

import jax
import jax.numpy as jnp
from jax import lax
from jax.experimental import pallas as pl
from jax.experimental.pallas import tpu as pltpu
from jax.experimental.pallas import tpu_sc as plsc

N = 10000
E = 320000
D_IN = 128
D_HID = 128
D_OUT = 64
N_CLASSES = 5
VCDN_HID = 128

NC = 2
NS = 16
NW = NC * NS
CH = 128
E_PT = 10240
NCH = E_PT // CH
E_PAD = E_PT * NW
N_ACC = 10112
R_S = N_ACC // NS

BLK = 400
NBLK = N // BLK

_mesh = plsc.VectorSubcoreMesh(core_axis_name="c", subcore_axis_name="s")
_sc_params = pltpu.CompilerParams(use_tc_tiling_on_sc=False)


@functools.partial(
    pl.kernel,
    out_type=jax.ShapeDtypeStruct((NC, 3, N_ACC, 16), jnp.float32),
    mesh=_mesh,
    compiler_params=_sc_params,
    scratch_types=[
        pltpu.VMEM_SHARED((3, N_ACC, 16), jnp.float32),
        pltpu.VMEM((CH,), jnp.int32),
        pltpu.VMEM((CH, 16), jnp.float32),
        pltpu.VMEM((R_S, 16), jnp.float32),
        pltpu.SemaphoreType.DMA,
    ],
)
def _deg_kernel(dst0_hbm, dst1_hbm, dst2_hbm, zeros_hbm, ones_hbm, out_hbm,
                acc, idx_v, ones_v, wb, sem):
    c = lax.axis_index("c")
    s = lax.axis_index("s")
    w = s * NC + c
    r0 = s * R_S
    pltpu.sync_copy(ones_hbm, ones_v)
    for o in range(3):
        pltpu.sync_copy(zeros_hbm.at[pl.ds(r0, R_S)], acc.at[o, pl.ds(r0, R_S)])
    plsc.subcore_barrier()
    for o, dst_hbm in enumerate((dst0_hbm, dst1_hbm, dst2_hbm)):
        def chunk(i, _, o=o, dst_hbm=dst_hbm):
            pltpu.sync_copy(dst_hbm.at[pl.ds(w * E_PT + i * CH, CH)], idx_v)
            pltpu.sync_copy(ones_v, acc.at[o].at[idx_v], add=True)
            return _
        lax.fori_loop(0, NCH, chunk, None)
    plsc.subcore_barrier()
    for o in range(3):
        pltpu.sync_copy(acc.at[o, pl.ds(r0, R_S)], wb)
        pltpu.sync_copy(wb, out_hbm.at[c, o, pl.ds(r0, R_S)])


@functools.partial(
    pl.kernel,
    out_type=jax.ShapeDtypeStruct((NC, N_ACC, D_OUT), jnp.float32),
    mesh=_mesh,
    compiler_params=_sc_params,
    scratch_types=[
        pltpu.VMEM_SHARED((N_ACC, D_OUT), jnp.float32),
        pltpu.VMEM((CH,), jnp.int32),
        pltpu.VMEM((CH,), jnp.int32),
        pltpu.VMEM((CH, D_OUT), jnp.float32),
        pltpu.VMEM((R_S, D_OUT), jnp.float32),
        pltpu.SemaphoreType.DMA,
    ],
)
def _msg_kernel(hs_hbm, src_hbm, dst_hbm, zeros_hbm, out_hbm,
                acc, idx_s, idx_d, rows, wb, sem):
    c = lax.axis_index("c")
    s = lax.axis_index("s")
    w = s * NC + c
    r0 = s * R_S
    base = w * E_PT
    pltpu.sync_copy(zeros_hbm.at[pl.ds(r0, R_S)], acc.at[pl.ds(r0, R_S)])
    plsc.subcore_barrier()

    def chunk(i, _):
        pltpu.sync_copy(src_hbm.at[pl.ds(base + i * CH, CH)], idx_s)
        pltpu.sync_copy(dst_hbm.at[pl.ds(base + i * CH, CH)], idx_d)
        pltpu.async_copy(hs_hbm.at[idx_s], rows, sem).wait()
        pltpu.sync_copy(rows, acc.at[idx_d], add=True)
        return _
    lax.fori_loop(0, NCH, chunk, None)
    plsc.subcore_barrier()
    pltpu.sync_copy(acc.at[pl.ds(r0, R_S)], wb)
    pltpu.sync_copy(wb, out_hbm.at[c, pl.ds(r0, R_S)])


def _mm(a, b):
    return jnp.dot(a, b, preferred_element_type=jnp.float32)


def _tc_stage1(x_ref, w1_ref, dinv_ref, oa_ref, ob_ref):
    r = _mm(x_ref[...], w1_ref[...]) * dinv_ref[...]
    oa_ref[...] = r[:, :D_OUT]
    ob_ref[...] = r[:, D_OUT:]


def _stage1(x, W1, dinv):
    return pl.pallas_call(
        _tc_stage1,
        grid=(NBLK,),
        in_specs=[
            pl.BlockSpec((BLK, D_IN), lambda i: (i, 0)),
            pl.BlockSpec((D_IN, D_HID), lambda i: (0, 0)),
            pl.BlockSpec((BLK, 1), lambda i: (i, 0)),
        ],
        out_specs=[
            pl.BlockSpec((BLK, D_OUT), lambda i: (i, 0)),
            pl.BlockSpec((BLK, D_OUT), lambda i: (i, 0)),
        ],
        out_shape=[
            jax.ShapeDtypeStruct((N, D_OUT), jnp.float32),
            jax.ShapeDtypeStruct((N, D_OUT), jnp.float32),
        ],
    )(x, W1, dinv)


def _tc_stage2(ma0_ref, ma1_ref, mb0_ref, mb1_ref, ha_ref, hb_ref,
               dinv_ref, b1_ref, w2_ref, o_ref):
    dinv = dinv_ref[...]
    b1 = b1_ref[...]
    ta = dinv * (ma0_ref[0] + ma1_ref[0] + ha_ref[...]) + b1[:, :D_OUT]
    tb = dinv * (mb0_ref[0] + mb1_ref[0] + hb_ref[...]) + b1[:, D_OUT:]
    t = jnp.maximum(jnp.concatenate([ta, tb], axis=1), 0.0)
    o_ref[...] = _mm(t, w2_ref[...]) * dinv


def _stage2(ma, mb, ha, hb, dinv, b1, W2):
    return pl.pallas_call(
        _tc_stage2,
        grid=(NBLK,),
        in_specs=[
            pl.BlockSpec((1, BLK, D_OUT), lambda i: (0, i, 0)),
            pl.BlockSpec((1, BLK, D_OUT), lambda i: (1, i, 0)),
            pl.BlockSpec((1, BLK, D_OUT), lambda i: (0, i, 0)),
            pl.BlockSpec((1, BLK, D_OUT), lambda i: (1, i, 0)),
            pl.BlockSpec((BLK, D_OUT), lambda i: (i, 0)),
            pl.BlockSpec((BLK, D_OUT), lambda i: (i, 0)),
            pl.BlockSpec((BLK, 1), lambda i: (i, 0)),
            pl.BlockSpec((1, D_HID), lambda i: (0, 0)),
            pl.BlockSpec((D_HID, D_OUT), lambda i: (0, 0)),
        ],
        out_specs=pl.BlockSpec((BLK, D_OUT), lambda i: (i, 0)),
        out_shape=jax.ShapeDtypeStruct((N, D_OUT), jnp.float32),
    )(ma, ma, mb, mb, ha, hb, dinv, b1, W2)


def _tc_final(m0a_ref, m0b_ref, h0_ref, d0_ref, b0_ref,
              m1a_ref, m1b_ref, h1_ref, d1_ref, b1_ref,
              m2a_ref, m2b_ref, h2_ref, d2_ref, b2_ref,
              wi1_ref, bi1_ref, wi2_ref, bi2_ref, o_ref):
    outs = []
    for ma, mb, h, d, b in ((m0a_ref, m0b_ref, h0_ref, d0_ref, b0_ref),
                            (m1a_ref, m1b_ref, h1_ref, d1_ref, b1_ref),
                            (m2a_ref, m2b_ref, h2_ref, d2_ref, b2_ref)):
        outs.append(d[...] * (ma[0] + mb[0] + h[...]) + b[...])
    flat = jnp.concatenate(outs, axis=1)
    t = jnp.maximum(_mm(flat, wi1_ref[...]) + bi1_ref[...], 0.0)
    o_ref[...] = _mm(t, wi2_ref[...]) + bi2_ref[...]


def _final(parts, Wi1, bi1, Wi2, bi2):
    in_specs = []
    args = []
    for msum, hs2, dinv, b2 in parts:
        in_specs += [
            pl.BlockSpec((1, BLK, D_OUT), lambda i: (0, i, 0)),
            pl.BlockSpec((1, BLK, D_OUT), lambda i: (1, i, 0)),
            pl.BlockSpec((BLK, D_OUT), lambda i: (i, 0)),
            pl.BlockSpec((BLK, 1), lambda i: (i, 0)),
            pl.BlockSpec((1, D_OUT), lambda i: (0, 0)),
        ]
        args += [msum, msum, hs2, dinv, b2]
    in_specs += [
        pl.BlockSpec((3 * D_OUT, VCDN_HID), lambda i: (0, 0)),
        pl.BlockSpec((1, VCDN_HID), lambda i: (0, 0)),
        pl.BlockSpec((VCDN_HID, N_CLASSES), lambda i: (0, 0)),
        pl.BlockSpec((1, N_CLASSES), lambda i: (0, 0)),
    ]
    args += [Wi1, bi1, Wi2, bi2]
    return pl.pallas_call(
        _tc_final,
        grid=(NBLK,),
        in_specs=in_specs,
        out_specs=pl.BlockSpec((BLK, N_CLASSES), lambda i: (i, 0)),
        out_shape=jax.ShapeDtypeStruct((N, N_CLASSES), jnp.float32),
    )(*args)


def kernel(x_mrna, edge_index_mrna, W1_mrna, b1_mrna, W2_mrna, b2_mrna,
           x_meth, edge_index_meth, W1_meth, b1_meth, W2_meth, b2_meth,
           x_mirna, edge_index_mirna, W1_mirna, b1_mirna, W2_mirna, b2_mirna,
           Wi1, bi1, Wi2, bi2):
    omics = [
        (x_mrna, edge_index_mrna, W1_mrna, b1_mrna, W2_mrna, b2_mrna),
        (x_meth, edge_index_meth, W1_meth, b1_meth, W2_meth, b2_meth),
        (x_mirna, edge_index_mirna, W1_mirna, b1_mirna, W2_mirna, b2_mirna),
    ]
    pad = E_PAD - E
    srcs, dsts = [], []
    for (_, ei, _, _, _, _) in omics:
        srcs.append(jnp.concatenate([ei[0], jnp.zeros((pad,), jnp.int32)]))
        dsts.append(jnp.concatenate([ei[1], jnp.full((pad,), N, jnp.int32)]))

    zeros16 = jnp.zeros((N_ACC, 16), jnp.float32)
    ones16 = jnp.ones((CH, 16), jnp.float32)
    zeros64 = jnp.zeros((N_ACC, D_OUT), jnp.float32)

    cnt = _deg_kernel(dsts[0], dsts[1], dsts[2], zeros16, ones16)
    deg = cnt[0, :, :N, 0] + cnt[1, :, :N, 0] + 1.0
    dinv_all = lax.rsqrt(deg)

    parts = []
    for o, (x, _, W1, b1, W2, b2) in enumerate(omics):
        dinv = dinv_all[o].reshape(N, 1)
        hs1a, hs1b = _stage1(x, W1, dinv)
        m1a = _msg_kernel(hs1a, srcs[o], dsts[o], zeros64)
        m1b = _msg_kernel(hs1b, srcs[o], dsts[o], zeros64)
        hs2 = _stage2(m1a, m1b, hs1a, hs1b, dinv, b1.reshape(1, D_HID), W2)
        m2 = _msg_kernel(hs2, srcs[o], dsts[o], zeros64)
        parts.append((m2, hs2, dinv, b2.reshape(1, D_OUT)))

    return _final(parts, Wi1, bi1.reshape(1, VCDN_HID), Wi2,
                  bi2.reshape(1, N_CLASSES))


# speedup vs baseline: 9.4378x; 1.4601x over previous
import functools

import jax
import jax.numpy as jnp
from jax import lax
from jax.experimental import pallas as pl
from jax.experimental.pallas import tpu as pltpu
from jax.experimental.pallas import tpu_sc as plsc

N = 10000
E = 320000
D_IN = 128
D_HID = 128
D_OUT = 64
N_CLASSES = 5
VCDN_HID = 128

NC = 2
NS = 16
NW = NC * NS
CH = 128
E_PT = 10240
NCH = E_PT // CH
E_PAD = E_PT * NW
N_ACC = 10112
R_S = N_ACC // NS

BLK = 400
NBLK = N // BLK

_mesh = plsc.VectorSubcoreMesh(core_axis_name="c", subcore_axis_name="s")
_sc_params = pltpu.CompilerParams(use_tc_tiling_on_sc=False)


DEG_K = 8


@functools.partial(
    pl.kernel,
    out_type=jax.ShapeDtypeStruct((NC, 3, N_ACC, 16), jnp.float32),
    mesh=_mesh,
    compiler_params=_sc_params,
    scratch_types=[
        pltpu.VMEM_SHARED((3, N_ACC, 16), jnp.float32),
        pltpu.VMEM((3, NCH, CH), jnp.int32),
        pltpu.VMEM((CH, 16), jnp.float32),
        pltpu.VMEM((R_S, 16), jnp.float32),
        pltpu.SemaphoreType.DMA,
    ],
)
def _deg_kernel(dst0_hbm, dst1_hbm, dst2_hbm, zeros_hbm, ones_hbm, out_hbm,
                acc, ida, ones_v, wb, sem):
    c = lax.axis_index("c")
    s = lax.axis_index("s")
    w = s * NC + c
    r0 = s * R_S
    pltpu.sync_copy(ones_hbm, ones_v)
    for o, dst_hbm in enumerate((dst0_hbm, dst1_hbm, dst2_hbm)):
        pltpu.sync_copy(dst_hbm.at[w], ida.at[o])
        pltpu.sync_copy(zeros_hbm.at[pl.ds(r0, R_S)], acc.at[o, pl.ds(r0, R_S)])
    plsc.subcore_barrier()
    for o in range(3):
        def group(g, _, o=o):
            for t in range(DEG_K):
                pltpu.async_copy(
                    ones_v, acc.at[o].at[ida.at[o, g * DEG_K + t]], sem,
                    add=True)
            for t in range(DEG_K):
                pltpu.make_async_copy(
                    ones_v, acc.at[o].at[ida.at[o, g * DEG_K + t]], sem).wait()
            return _
        lax.fori_loop(0, NCH // DEG_K, group, None)
    plsc.subcore_barrier()
    for o in range(3):
        pltpu.sync_copy(acc.at[o, pl.ds(r0, R_S)], wb)
        pltpu.sync_copy(wb, out_hbm.at[c, o, pl.ds(r0, R_S)])


@functools.partial(
    pl.kernel,
    out_type=jax.ShapeDtypeStruct((NC, N_ACC, D_OUT), jnp.float32),
    mesh=_mesh,
    compiler_params=_sc_params,
    scratch_types=[
        pltpu.VMEM_SHARED((N_ACC, D_OUT), jnp.float32),
        pltpu.VMEM((NCH, CH), jnp.int32),
        pltpu.VMEM((NCH, CH), jnp.int32),
        pltpu.VMEM((CH, D_OUT), jnp.float32),
        pltpu.VMEM((CH, D_OUT), jnp.float32),
        pltpu.VMEM((R_S, D_OUT), jnp.float32),
        pltpu.SemaphoreType.DMA,
        pltpu.SemaphoreType.DMA,
    ],
)
def _msg_kernel(hs_hbm, src_hbm, dst_hbm, zeros_hbm, out_hbm,
                acc, isa, ida, rows0, rows1, wb, sem0, sem1):
    c = lax.axis_index("c")
    s = lax.axis_index("s")
    w = s * NC + c
    r0 = s * R_S
    pltpu.sync_copy(src_hbm.at[w], isa)
    pltpu.sync_copy(dst_hbm.at[w], ida)
    pltpu.sync_copy(zeros_hbm.at[pl.ds(r0, R_S)], acc.at[pl.ds(r0, R_S)])
    plsc.subcore_barrier()

    pltpu.async_copy(hs_hbm.at[isa.at[0]], rows0, sem0)

    def pair(j, _):
        c1 = 2 * j + 1
        pltpu.async_copy(hs_hbm.at[isa.at[c1]], rows1, sem1)
        pltpu.make_async_copy(hs_hbm.at[isa.at[c1 - 1]], rows0, sem0).wait()
        pltpu.sync_copy(rows0, acc.at[ida.at[c1 - 1]], add=True)

        @pl.when(c1 + 1 < NCH)
        def _start_next():
            pltpu.async_copy(hs_hbm.at[isa.at[c1 + 1]], rows0, sem0)
        pltpu.make_async_copy(hs_hbm.at[isa.at[c1]], rows1, sem1).wait()
        pltpu.sync_copy(rows1, acc.at[ida.at[c1]], add=True)
        return _
    lax.fori_loop(0, NCH // 2, pair, None)
    plsc.subcore_barrier()
    pltpu.sync_copy(acc.at[pl.ds(r0, R_S)], wb)
    pltpu.sync_copy(wb, out_hbm.at[c, pl.ds(r0, R_S)])


def _mm(a, b):
    return jnp.dot(a, b, preferred_element_type=jnp.float32)


def _tc_stage1(x_ref, w1_ref, dinv_ref, oa_ref, ob_ref):
    r = _mm(x_ref[...], w1_ref[...]) * dinv_ref[...]
    oa_ref[...] = r[:, :D_OUT]
    ob_ref[...] = r[:, D_OUT:]


def _stage1(x, W1, dinv):
    return pl.pallas_call(
        _tc_stage1,
        grid=(NBLK,),
        in_specs=[
            pl.BlockSpec((BLK, D_IN), lambda i: (i, 0)),
            pl.BlockSpec((D_IN, D_HID), lambda i: (0, 0)),
            pl.BlockSpec((BLK, 1), lambda i: (i, 0)),
        ],
        out_specs=[
            pl.BlockSpec((BLK, D_OUT), lambda i: (i, 0)),
            pl.BlockSpec((BLK, D_OUT), lambda i: (i, 0)),
        ],
        out_shape=[
            jax.ShapeDtypeStruct((N, D_OUT), jnp.float32),
            jax.ShapeDtypeStruct((N, D_OUT), jnp.float32),
        ],
    )(x, W1, dinv)


def _tc_stage2(ma0_ref, ma1_ref, mb0_ref, mb1_ref, ha_ref, hb_ref,
               dinv_ref, b1_ref, w2_ref, o_ref):
    dinv = dinv_ref[...]
    b1 = b1_ref[...]
    ta = dinv * (ma0_ref[0] + ma1_ref[0] + ha_ref[...]) + b1[:, :D_OUT]
    tb = dinv * (mb0_ref[0] + mb1_ref[0] + hb_ref[...]) + b1[:, D_OUT:]
    t = jnp.maximum(jnp.concatenate([ta, tb], axis=1), 0.0)
    o_ref[...] = _mm(t, w2_ref[...]) * dinv


def _stage2(ma, mb, ha, hb, dinv, b1, W2):
    return pl.pallas_call(
        _tc_stage2,
        grid=(NBLK,),
        in_specs=[
            pl.BlockSpec((1, BLK, D_OUT), lambda i: (0, i, 0)),
            pl.BlockSpec((1, BLK, D_OUT), lambda i: (1, i, 0)),
            pl.BlockSpec((1, BLK, D_OUT), lambda i: (0, i, 0)),
            pl.BlockSpec((1, BLK, D_OUT), lambda i: (1, i, 0)),
            pl.BlockSpec((BLK, D_OUT), lambda i: (i, 0)),
            pl.BlockSpec((BLK, D_OUT), lambda i: (i, 0)),
            pl.BlockSpec((BLK, 1), lambda i: (i, 0)),
            pl.BlockSpec((1, D_HID), lambda i: (0, 0)),
            pl.BlockSpec((D_HID, D_OUT), lambda i: (0, 0)),
        ],
        out_specs=pl.BlockSpec((BLK, D_OUT), lambda i: (i, 0)),
        out_shape=jax.ShapeDtypeStruct((N, D_OUT), jnp.float32),
    )(ma, ma, mb, mb, ha, hb, dinv, b1, W2)


def _tc_final(m0a_ref, m0b_ref, h0_ref, d0_ref, b0_ref,
              m1a_ref, m1b_ref, h1_ref, d1_ref, b1_ref,
              m2a_ref, m2b_ref, h2_ref, d2_ref, b2_ref,
              wi1_ref, bi1_ref, wi2_ref, bi2_ref, o_ref):
    outs = []
    for ma, mb, h, d, b in ((m0a_ref, m0b_ref, h0_ref, d0_ref, b0_ref),
                            (m1a_ref, m1b_ref, h1_ref, d1_ref, b1_ref),
                            (m2a_ref, m2b_ref, h2_ref, d2_ref, b2_ref)):
        outs.append(d[...] * (ma[0] + mb[0] + h[...]) + b[...])
    flat = jnp.concatenate(outs, axis=1)
    t = jnp.maximum(_mm(flat, wi1_ref[...]) + bi1_ref[...], 0.0)
    o_ref[...] = _mm(t, wi2_ref[...]) + bi2_ref[...]


def _final(parts, Wi1, bi1, Wi2, bi2):
    in_specs = []
    args = []
    for msum, hs2, dinv, b2 in parts:
        in_specs += [
            pl.BlockSpec((1, BLK, D_OUT), lambda i: (0, i, 0)),
            pl.BlockSpec((1, BLK, D_OUT), lambda i: (1, i, 0)),
            pl.BlockSpec((BLK, D_OUT), lambda i: (i, 0)),
            pl.BlockSpec((BLK, 1), lambda i: (i, 0)),
            pl.BlockSpec((1, D_OUT), lambda i: (0, 0)),
        ]
        args += [msum, msum, hs2, dinv, b2]
    in_specs += [
        pl.BlockSpec((3 * D_OUT, VCDN_HID), lambda i: (0, 0)),
        pl.BlockSpec((1, VCDN_HID), lambda i: (0, 0)),
        pl.BlockSpec((VCDN_HID, N_CLASSES), lambda i: (0, 0)),
        pl.BlockSpec((1, N_CLASSES), lambda i: (0, 0)),
    ]
    args += [Wi1, bi1, Wi2, bi2]
    return pl.pallas_call(
        _tc_final,
        grid=(NBLK,),
        in_specs=in_specs,
        out_specs=pl.BlockSpec((BLK, N_CLASSES), lambda i: (i, 0)),
        out_shape=jax.ShapeDtypeStruct((N, N_CLASSES), jnp.float32),
    )(*args)


def kernel(x_mrna, edge_index_mrna, W1_mrna, b1_mrna, W2_mrna, b2_mrna,
           x_meth, edge_index_meth, W1_meth, b1_meth, W2_meth, b2_meth,
           x_mirna, edge_index_mirna, W1_mirna, b1_mirna, W2_mirna, b2_mirna,
           Wi1, bi1, Wi2, bi2):
    omics = [
        (x_mrna, edge_index_mrna, W1_mrna, b1_mrna, W2_mrna, b2_mrna),
        (x_meth, edge_index_meth, W1_meth, b1_meth, W2_meth, b2_meth),
        (x_mirna, edge_index_mirna, W1_mirna, b1_mirna, W2_mirna, b2_mirna),
    ]
    pad = E_PAD - E
    srcs, dsts = [], []
    for (_, ei, _, _, _, _) in omics:
        srcs.append(jnp.concatenate(
            [ei[0], jnp.zeros((pad,), jnp.int32)]).reshape(NW, NCH, CH))
        dsts.append(jnp.concatenate(
            [ei[1], jnp.full((pad,), N, jnp.int32)]).reshape(NW, NCH, CH))

    zeros16 = jnp.zeros((N_ACC, 16), jnp.float32)
    ones16 = jnp.ones((CH, 16), jnp.float32)
    zeros64 = jnp.zeros((N_ACC, D_OUT), jnp.float32)

    cnt = _deg_kernel(dsts[0], dsts[1], dsts[2], zeros16, ones16)
    deg = cnt[0, :, :N, 0] + cnt[1, :, :N, 0] + 1.0
    dinv_all = lax.rsqrt(deg)

    parts = []
    for o, (x, _, W1, b1, W2, b2) in enumerate(omics):
        dinv = dinv_all[o].reshape(N, 1)
        hs1a, hs1b = _stage1(x, W1, dinv)
        m1a = _msg_kernel(hs1a, srcs[o], dsts[o], zeros64)
        m1b = _msg_kernel(hs1b, srcs[o], dsts[o], zeros64)
        hs2 = _stage2(m1a, m1b, hs1a, hs1b, dinv, b1.reshape(1, D_HID), W2)
        m2 = _msg_kernel(hs2, srcs[o], dsts[o], zeros64)
        parts.append((m2, hs2, dinv, b2.reshape(1, D_OUT)))

    return _final(parts, Wi1, bi1.reshape(1, VCDN_HID), Wi2,
                  bi2.reshape(1, N_CLASSES))
